# Initial kernel scaffold; baseline (speedup 1.0000x reference)
import jax
import jax.numpy as jnp
from jax.experimental import pallas as pl


def kernel(x, edge_index, edge_weight, W0, gru0_w_ih, gru0_w_hh, gru0_b_ih, gru0_b_hh, lin0_w, lin0_b, W1, gru1_w_ih, gru1_w_hh, gru1_b_ih, gru1_b_hh, lin1_w, lin1_b):
    raise NotImplementedError("write your pallas kernel here")


# speedup vs baseline: 9.1500x; 9.1500x over previous
import functools

import jax
import jax.numpy as jnp
from jax import lax
from jax.experimental import pallas as pl
from jax.experimental.pallas import tpu as pltpu
from jax.experimental.pallas import tpu_sc as plsc

N = 10000
E = 320000
D = 128
NP = 10240
HALF = NP // 2
NC = 2
NS = 16
NW = NC * NS
L = 16
CH = 128
T = 79
EPW = T * CH
EP = NW * EPW
ROWS_PER_TILE = NP // NS

_mesh = plsc.VectorSubcoreMesh(core_axis_name="c", subcore_axis_name="s")
_HI = lax.Precision.HIGHEST


@functools.partial(
    pl.kernel,
    mesh=_mesh,
    out_type=jax.ShapeDtypeStruct((NW, NP * L), jnp.float32),
    scratch_types=[
        pltpu.VMEM((EPW,), jnp.int32),
        pltpu.VMEM((EPW,), jnp.float32),
        pltpu.VMEM((HALF * L,), jnp.float32),
    ],
    compiler_params=pltpu.CompilerParams(needs_layout_passes=False),
)
def _deg_sc(dst_hbm, ew_hbm, out_hbm, dst_v, ew_v, degw):
    c = lax.axis_index("c")
    s = lax.axis_index("s")
    w = c * NS + s
    pltpu.sync_copy(dst_hbm.at[w], dst_v)
    pltpu.sync_copy(ew_hbm.at[w], ew_v)
    col = lax.iota(jnp.int32, L)
    for half in range(2):
        lo = half * HALF

        def zbody(i, carry):
            degw[pl.ds(i * L, L)] = jnp.zeros((L,), jnp.float32)
            return carry

        lax.fori_loop(0, HALF, zbody, 0)

        def ebody(g, carry):
            d = dst_v[pl.ds(g * L, L)]
            wv = ew_v[pl.ds(g * L, L)]
            idx = (d - lo) * L + col
            m = (d >= lo) & (d < lo + HALF)
            plsc.addupdate_scatter(degw, [idx], wv, mask=m)
            return carry

        lax.fori_loop(0, EPW // L, ebody, 0)
        pltpu.sync_copy(degw, out_hbm.at[w, pl.ds(lo * L, HALF * L)])


@functools.partial(
    pl.kernel,
    mesh=_mesh,
    out_type=jax.ShapeDtypeStruct((NC, NP, D), jnp.float32),
    scratch_types=[
        pltpu.VMEM((T, CH), jnp.int32),
        pltpu.VMEM((T, CH), jnp.int32),
        pltpu.VMEM((EPW,), jnp.float32),
        pltpu.VMEM((CH, D), jnp.float32),
        pltpu.VMEM_SHARED((NP, D), jnp.float32),
        pltpu.SemaphoreType.DMA,
    ],
    compiler_params=pltpu.CompilerParams(needs_layout_passes=False),
)
def _edge_sc(y_hbm, src_hbm, dst_hbm, ew_hbm, out_hbm,
             src_v, dst_v, ew_v, rows_v, acc_sh, gsem):
    c = lax.axis_index("c")
    s = lax.axis_index("s")
    w = c * NS + s

    def zbody(i, carry):
        for f in range(D // L):
            rows_v[i, pl.ds(f * L, L)] = jnp.zeros((L,), jnp.float32)
        return carry

    lax.fori_loop(0, CH, zbody, 0)
    for k in range(ROWS_PER_TILE // CH):
        pltpu.sync_copy(rows_v, acc_sh.at[pl.ds(s * ROWS_PER_TILE + k * CH, CH)])

    pltpu.sync_copy(src_hbm.at[w], src_v)
    pltpu.sync_copy(dst_hbm.at[w], dst_v)
    pltpu.sync_copy(ew_hbm.at[w], ew_v)
    plsc.subcore_barrier()

    def chunk(j, carry):
        pltpu.async_copy(y_hbm.at[src_v.at[j]], rows_v, gsem).wait()

        def ebody(e, icarry):
            wv = plsc.load_gather(ew_v, [jnp.full((L,), j * CH + e, jnp.int32)])
            for f in range(D // L):
                sl = pl.ds(f * L, L)
                rows_v[e, sl] = rows_v[e, sl] * wv
            return icarry

        lax.fori_loop(0, CH, ebody, 0)
        pltpu.sync_copy(rows_v, acc_sh.at[dst_v.at[j]], add=True)
        return carry

    lax.fori_loop(0, T, chunk, 0)
    plsc.subcore_barrier()
    pltpu.sync_copy(acc_sh.at[pl.ds(s * ROWS_PER_TILE, ROWS_PER_TILE)],
                    out_hbm.at[c, pl.ds(s * ROWS_PER_TILE, ROWS_PER_TILE)])


def _gru_math(W, wih, whh, bih, bhh):
    gx = lax.dot_general(W, wih, (((1,), (1,)), ((), ())), precision=_HI)
    gx = gx + bih[None, :]
    gh = lax.dot_general(W, whh, (((1,), (1,)), ((), ())), precision=_HI)
    gh = gh + bhh[None, :]
    d = W.shape[1]
    r = jax.nn.sigmoid(gx[:, :d] + gh[:, :d])
    z = jax.nn.sigmoid(gx[:, d:2 * d] + gh[:, d:2 * d])
    n = jnp.tanh(gx[:, 2 * d:] + r * gh[:, 2 * d:])
    return (1.0 - z) * n + z * W


def _gru_body(W0r, wih0, whh0, bih0, bhh0, W1r, wih1, whh1, bih1, bhh1,
              Wa_ref, Wb_ref):
    Wa_ref[...] = _gru_math(W0r[...], wih0[...], whh0[...], bih0[...], bhh0[...])
    Wb_ref[...] = _gru_math(W1r[...], wih1[...], whh1[...], bih1[...], bhh1[...])


def _gru_call(W0, g0wi, g0wh, g0bi, g0bh, W1, g1wi, g1wh, g1bi, g1bh):
    return pl.pallas_call(
        _gru_body,
        out_shape=(jax.ShapeDtypeStruct((D, D), jnp.float32),
                   jax.ShapeDtypeStruct((D, D), jnp.float32)),
    )(W0, g0wi, g0wh, g0bi, g0bh, W1, g1wi, g1wh, g1bi, g1bh)


_BLK = 1024
_G = NP // _BLK


def _prep_body(degp_ref, x_ref, Wa_ref, y0_ref, dinv_ref):
    degp = degp_ref[...].reshape(NW, _BLK, L)
    deg = jnp.sum(degp, axis=(0, 2)) + 1.0
    dinv = lax.rsqrt(deg)
    xw = lax.dot_general(x_ref[...], Wa_ref[...], (((1,), (0,)), ((), ())),
                         precision=_HI)
    y0_ref[...] = xw * dinv[:, None]
    dinv_ref[...] = dinv


def _prep_call(degp, x_p, Wa):
    return pl.pallas_call(
        _prep_body,
        grid=(_G,),
        in_specs=[
            pl.BlockSpec((NW, _BLK * L), lambda i: (0, i)),
            pl.BlockSpec((_BLK, D), lambda i: (i, 0)),
            pl.BlockSpec((D, D), lambda i: (0, 0)),
        ],
        out_specs=[
            pl.BlockSpec((_BLK, D), lambda i: (i, 0)),
            pl.BlockSpec((_BLK,), lambda i: (i,)),
        ],
        out_shape=(jax.ShapeDtypeStruct((NP, D), jnp.float32),
                   jax.ShapeDtypeStruct((NP,), jnp.float32)),
    )(degp, x_p, Wa)


def _mid_body(a_ref, y0_ref, dinv_ref, l0w_ref, l0b_ref, Wb_ref, y1_ref):
    dinv = dinv_ref[...][:, None]
    t = (a_ref[0] + a_ref[1] + y0_ref[...]) * dinv
    h = jnp.maximum(t, 0.0)
    h1 = lax.dot_general(h, l0w_ref[...], (((1,), (1,)), ((), ())),
                         precision=_HI) + l0b_ref[...][None, :]
    y1_ref[...] = lax.dot_general(h1, Wb_ref[...], (((1,), (0,)), ((), ())),
                                  precision=_HI) * dinv


def _mid_call(acc, y0, dinv, l0w, l0b, Wb):
    return pl.pallas_call(
        _mid_body,
        grid=(_G,),
        in_specs=[
            pl.BlockSpec((NC, _BLK, D), lambda i: (0, i, 0)),
            pl.BlockSpec((_BLK, D), lambda i: (i, 0)),
            pl.BlockSpec((_BLK,), lambda i: (i,)),
            pl.BlockSpec((D, D), lambda i: (0, 0)),
            pl.BlockSpec((D,), lambda i: (0,)),
            pl.BlockSpec((D, D), lambda i: (0, 0)),
        ],
        out_specs=pl.BlockSpec((_BLK, D), lambda i: (i, 0)),
        out_shape=jax.ShapeDtypeStruct((NP, D), jnp.float32),
    )(acc, y0, dinv, l0w, l0b, Wb)


def _fin_body(a_ref, y1_ref, dinv_ref, l1w_ref, l1b_ref, o_ref):
    dinv = dinv_ref[...][:, None]
    t = (a_ref[0] + a_ref[1] + y1_ref[...]) * dinv
    o = lax.dot_general(t, l1w_ref[...], (((1,), (1,)), ((), ())),
                        precision=_HI) + l1b_ref[...][None, :]
    o_ref[...] = jax.nn.sigmoid(o)


def _fin_call(acc, y1, dinv, l1w_p, l1b_p):
    return pl.pallas_call(
        _fin_body,
        grid=(_G,),
        in_specs=[
            pl.BlockSpec((NC, _BLK, D), lambda i: (0, i, 0)),
            pl.BlockSpec((_BLK, D), lambda i: (i, 0)),
            pl.BlockSpec((_BLK,), lambda i: (i,)),
            pl.BlockSpec((D, D), lambda i: (0, 0)),
            pl.BlockSpec((D,), lambda i: (0,)),
        ],
        out_specs=pl.BlockSpec((_BLK, D), lambda i: (i, 0)),
        out_shape=jax.ShapeDtypeStruct((NP, D), jnp.float32),
    )(acc, y1, dinv, l1w_p, l1b_p)


def kernel(x, edge_index, edge_weight, W0, gru0_w_ih, gru0_w_hh, gru0_b_ih,
           gru0_b_hh, lin0_w, lin0_b, W1, gru1_w_ih, gru1_w_hh, gru1_b_ih,
           gru1_b_hh, lin1_w, lin1_b):
    src = edge_index[0].astype(jnp.int32)
    dst = edge_index[1].astype(jnp.int32)
    pad = EP - E
    src_p = jnp.concatenate([src, jnp.zeros((pad,), jnp.int32)])
    dst_p = jnp.concatenate([dst, jnp.zeros((pad,), jnp.int32)])
    ew_p = jnp.concatenate([edge_weight, jnp.zeros((pad,), jnp.float32)])
    src3 = src_p.reshape(NW, T, CH)
    dst3 = dst_p.reshape(NW, T, CH)
    dst2 = dst_p.reshape(NW, EPW)
    ew2 = ew_p.reshape(NW, EPW)
    x_p = jnp.concatenate([x, jnp.zeros((NP - N, D), jnp.float32)])
    l1w_p = jnp.zeros((D, D), jnp.float32).at[: lin1_w.shape[0]].set(lin1_w)
    l1b_p = jnp.zeros((D,), jnp.float32).at[: lin1_b.shape[0]].set(lin1_b)

    Wa, Wb = _gru_call(W0, gru0_w_ih, gru0_w_hh, gru0_b_ih, gru0_b_hh,
                       W1, gru1_w_ih, gru1_w_hh, gru1_b_ih, gru1_b_hh)
    degp = _deg_sc(dst2, ew2)
    y0, dinv = _prep_call(degp, x_p, Wa)
    acc0 = _edge_sc(y0, src3, dst3, ew2)
    y1 = _mid_call(acc0, y0, dinv, lin0_w, lin0_b, Wb)
    acc1 = _edge_sc(y1, src3, dst3, ew2)
    o = _fin_call(acc1, y1, dinv, l1w_p, l1b_p)
    return o[:N, : lin1_w.shape[0]]
